# baseline (speedup 1.0000x reference)
import functools

import numpy as np
import jax
import jax.numpy as jnp
from jax import lax
from jax.experimental import pallas as pl
from jax.experimental.pallas import tpu as pltpu
from jax.experimental.pallas import tpu_sc as plsc

N = 10000
E = 320000
D = 128
DE = 16
PW = 8
NM = N // 2

NB = 5
BN = N // NB
EBG = 100
BEB = E // EBG

NTILES = 32
TILE_E = E // NTILES

MC = 40
MCP = 48
MNCH = TILE_E // MC
MG = MCP // 16

SB = 2000
NSB = TILE_E // SB
SGRP = SB // 16

_SCP = pltpu.CompilerParams(needs_layout_passes=False)
_MESH = plsc.VectorSubcoreMesh(core_axis_name="c", subcore_axis_name="s")


def _mask_constants():
    perm = jax.random.permutation(jax.random.key(42), N)
    mask_idx = perm[:NM]
    inv = jnp.argsort(perm)
    visf = (inv >= NM).astype(jnp.float32)
    pos_m = jax.random.normal(jax.random.key(7), (NM, 3), jnp.float32)
    pm_full = jnp.concatenate(
        [jnp.pad(pos_m, ((0, 0), (0, PW - 3))), jnp.zeros((NM, PW))], axis=0)
    posm8 = pm_full[inv]
    vis8 = jnp.pad(visf[:, None], ((0, 0), (0, PW - 1)))
    return mask_idx, visf, posm8, vis8


def _tc_node_pre(x_ref, w1_ref, w2_ref, a_ref, b_ref):
    xv = x_ref[...]
    a_ref[...] = jnp.dot(xv, w1_ref[...], preferred_element_type=jnp.float32)
    b_ref[...] = jnp.dot(xv, w2_ref[...], preferred_element_type=jnp.float32)


def _node_pre(x, w1, w2):
    return pl.pallas_call(
        _tc_node_pre,
        grid=(NB,),
        in_specs=[pl.BlockSpec((BN, D), lambda i: (i, 0)),
                  pl.BlockSpec((D, D), lambda i: (0, 0)),
                  pl.BlockSpec((D, D), lambda i: (0, 0))],
        out_specs=[pl.BlockSpec((BN, D), lambda i: (i, 0)),
                   pl.BlockSpec((BN, D), lambda i: (i, 0))],
        out_shape=[jax.ShapeDtypeStruct((N, D), jnp.float32),
                   jax.ShapeDtypeStruct((N, D), jnp.float32)],
    )(x, w1, w2)


def _tc_edge_pre(ea_ref, w3e_ref, w3d_ref, be_ref, bd_ref, oe_ref, od_ref):
    eav = ea_ref[...]
    oe_ref[...] = jnp.dot(eav, w3e_ref[...], preferred_element_type=jnp.float32) + be_ref[...]
    od_ref[...] = jnp.dot(eav, w3d_ref[...], preferred_element_type=jnp.float32) + bd_ref[...]


def _edge_pre(edge_attr, w3e, w3d, be, bd):
    return pl.pallas_call(
        _tc_edge_pre,
        grid=(EBG,),
        in_specs=[pl.BlockSpec((BEB, DE), lambda i: (i, 0)),
                  pl.BlockSpec((DE, D), lambda i: (0, 0)),
                  pl.BlockSpec((DE, D), lambda i: (0, 0)),
                  pl.BlockSpec((1, D), lambda i: (0, 0)),
                  pl.BlockSpec((1, D), lambda i: (0, 0))],
        out_specs=[pl.BlockSpec((BEB, D), lambda i: (i, 0)),
                   pl.BlockSpec((BEB, D), lambda i: (i, 0))],
        out_shape=[jax.ShapeDtypeStruct((E, D), jnp.float32),
                   jax.ShapeDtypeStruct((E, D), jnp.float32)],
    )(edge_attr, w3e, w3d, be, bd)


def _tc_reduce(p_ref, o_ref):
    o_ref[...] = jnp.sum(p_ref[...], axis=0, keepdims=True)


def _reduce_pacc(pacc_p):
    seg = PW * N // NB
    return pl.pallas_call(
        _tc_reduce,
        grid=(NB,),
        in_specs=[pl.BlockSpec((NTILES, seg), lambda i: (0, i))],
        out_specs=[pl.BlockSpec((1, seg), lambda i: (0, i))],
        out_shape=[jax.ShapeDtypeStruct((1, PW * N), jnp.float32)],
    )(pacc_p)[0].reshape(N, PW)


def _tc_enc_upd(x_ref, a0, a1, pacc_ref, pos_ref, vis_ref, posm_ref, tok_ref,
                wu1_ref, wu2_ref, bu_ref, w1d_ref, w2d_ref,
                a2_ref, b2_ref, pc_ref):
    agg = a0[...] + a1[...]
    pacc = pacc_ref[...]
    deg = pacc[:, 3:4] + 1.0
    h = jnp.maximum(
        jnp.dot(x_ref[...], wu1_ref[...], preferred_element_type=jnp.float32)
        + jnp.dot(agg / deg, wu2_ref[...], preferred_element_type=jnp.float32)
        + bu_ref[...], 0.0)
    vis = vis_ref[:, 0:1] > 0.0
    z = jnp.where(vis, h, tok_ref[...])
    pc_ref[...] = jnp.where(vis, pos_ref[...] + pacc / deg, posm_ref[...])
    a2_ref[...] = jnp.dot(z, w1d_ref[...], preferred_element_type=jnp.float32)
    b2_ref[...] = jnp.dot(z, w2d_ref[...], preferred_element_type=jnp.float32)


def _enc_upd(x, a0, a1, pacc, pos8, vis8, posm8, tok, wu1, wu2, bu, w1d, w2d):
    nd = lambda i: (i, 0)
    w0 = lambda i: (0, 0)
    return pl.pallas_call(
        _tc_enc_upd,
        grid=(NB,),
        in_specs=[pl.BlockSpec((BN, D), nd), pl.BlockSpec((BN, D), nd),
                  pl.BlockSpec((BN, D), nd), pl.BlockSpec((BN, PW), nd),
                  pl.BlockSpec((BN, PW), nd), pl.BlockSpec((BN, PW), nd),
                  pl.BlockSpec((BN, PW), nd),
                  pl.BlockSpec((1, D), w0),
                  pl.BlockSpec((D, D), w0), pl.BlockSpec((D, D), w0),
                  pl.BlockSpec((1, D), w0),
                  pl.BlockSpec((D, D), w0), pl.BlockSpec((D, D), w0)],
        out_specs=[pl.BlockSpec((BN, D), nd), pl.BlockSpec((BN, D), nd),
                   pl.BlockSpec((BN, PW), nd)],
        out_shape=[jax.ShapeDtypeStruct((N, D), jnp.float32),
                   jax.ShapeDtypeStruct((N, D), jnp.float32),
                   jax.ShapeDtypeStruct((N, PW), jnp.float32)],
    )(x, a0, a1, pacc, pos8, vis8, posm8, tok, wu1, wu2, bu, w1d, w2d)


def _tc_final(a0, a1, pacc_ref, tok_ref, wu1_ref, wu2_ref, bu_ref, wo_ref,
              bo_ref, posm_ref, out_ref):
    agg = a0[...] + a1[...]
    pacc = pacc_ref[...]
    deg = pacc[:, 3:4] + 1.0
    hz = jnp.maximum(
        jnp.dot(tok_ref[...], wu1_ref[...], preferred_element_type=jnp.float32)
        + jnp.dot(agg / deg, wu2_ref[...], preferred_element_type=jnp.float32)
        + bu_ref[...], 0.0)
    out_ref[...] = (jnp.dot(hz, wo_ref[...], preferred_element_type=jnp.float32)
                    + bo_ref[...] + posm_ref[...] + pacc / deg)


def _final(a0, a1, pacc, tok, wu1, wu2, bu, wo8, bo8, posm8):
    nd = lambda i: (i, 0)
    w0 = lambda i: (0, 0)
    return pl.pallas_call(
        _tc_final,
        grid=(NB,),
        in_specs=[pl.BlockSpec((BN, D), nd), pl.BlockSpec((BN, D), nd),
                  pl.BlockSpec((BN, PW), nd),
                  pl.BlockSpec((1, D), w0),
                  pl.BlockSpec((D, D), w0), pl.BlockSpec((D, D), w0),
                  pl.BlockSpec((1, D), w0),
                  pl.BlockSpec((D, PW), w0), pl.BlockSpec((1, PW), w0),
                  pl.BlockSpec((BN, PW), nd)],
        out_specs=[pl.BlockSpec((BN, PW), nd)],
        out_shape=[jax.ShapeDtypeStruct((N, PW), jnp.float32)],
    )(a0, a1, pacc, tok, wu1, wu2, bu, wo8, bo8, posm8)[0]


@functools.partial(
    pl.kernel,
    out_type=jax.ShapeDtypeStruct((E,), jnp.float32),
    mesh=_MESH,
    scratch_types=[
        pltpu.VMEM((N,), jnp.float32),
        pltpu.VMEM((N,), jnp.float32),
        pltpu.VMEM((N,), jnp.float32),
        pltpu.VMEM((N,), jnp.float32),
        pltpu.VMEM((N,), jnp.float32),
        pltpu.VMEM((SB,), jnp.int32),
        pltpu.VMEM((SB,), jnp.int32),
        pltpu.VMEM((SB,), jnp.float32),
    ],
    compiler_params=_SCP)
def _sc_pre(fs_h, fd_h, px_h, py_h, pz_h, src_h, dst_h, d2k_h,
            fs_v, fd_v, px_v, py_v, pz_v, sbig, dbig, obig):
    cid = lax.axis_index("c")
    sid = lax.axis_index("s")
    wid = cid * 16 + sid
    pltpu.sync_copy(fs_h, fs_v)
    pltpu.sync_copy(fd_h, fd_v)
    pltpu.sync_copy(px_h, px_v)
    pltpu.sync_copy(py_h, py_v)
    pltpu.sync_copy(pz_h, pz_v)

    def stage_body(t, carry):
        base = wid * TILE_E + t * SB
        pltpu.sync_copy(src_h.at[pl.ds(base, SB)], sbig)
        pltpu.sync_copy(dst_h.at[pl.ds(base, SB)], dbig)

        def grp(g, carry2):
            sv = sbig[pl.ds(g * 16, 16)]
            dv = dbig[pl.ds(g * 16, 16)]
            kf = plsc.load_gather(fs_v, [sv]) * plsc.load_gather(fd_v, [dv])
            dx = plsc.load_gather(px_v, [sv]) - plsc.load_gather(px_v, [dv])
            dy = plsc.load_gather(py_v, [sv]) - plsc.load_gather(py_v, [dv])
            dz = plsc.load_gather(pz_v, [sv]) - plsc.load_gather(pz_v, [dv])
            d2 = dx * dx + dy * dy + dz * dz
            obig[pl.ds(g * 16, 16)] = jnp.where(kf > 0.0, d2, -1.0)
            return carry2

        lax.fori_loop(0, SGRP, grp, 0)
        pltpu.sync_copy(obig, d2k_h.at[pl.ds(base, SB)])
        return carry

    lax.fori_loop(0, NSB, stage_body, 0)


_MAIN_OUT = [jax.ShapeDtypeStruct((N, D), jnp.float32),
             jax.ShapeDtypeStruct((N, D), jnp.float32),
             jax.ShapeDtypeStruct((E,), jnp.float32)]

_ZROWS = 624


@functools.partial(
    pl.kernel,
    out_type=_MAIN_OUT,
    mesh=_MESH,
    scratch_types=[
        pltpu.VMEM_SHARED((N, D), jnp.float32),
        [pltpu.VMEM((1, MCP), jnp.int32)] * 2,
        [pltpu.VMEM((1, MCP), jnp.int32)] * 2,
        [pltpu.VMEM((1, MCP), jnp.float32)] * 2,
        [pltpu.VMEM((1, MCP), jnp.float32)] * 2,
        [pltpu.VMEM((MCP, D), jnp.float32)] * 2,
        [pltpu.VMEM((MCP, D), jnp.float32)] * 2,
        [pltpu.VMEM((MCP, D), jnp.float32)] * 2,
        pltpu.VMEM((D * 16,), jnp.float32),
        pltpu.VMEM((D * 16,), jnp.float32),
        pltpu.VMEM((16, D), jnp.float32),
        [pltpu.SemaphoreType.DMA] * 2,
    ],
    compiler_params=_SCP)
def _sc_main(a_h, b_h, ea_h, d2k_h, src_h, dst_h, wr_h, wc_h,
             agg0_h, agg1_h, ct_h,
             agg_sh, sidx, didx, kbuf, cbuf, arows, brows, erows,
             wr_v, wc_v, zb, gsem):
    cid = lax.axis_index("c")
    sid = lax.axis_index("s")
    wid = cid * 16 + sid
    tbase = wid * TILE_E
    pltpu.sync_copy(wr_h, wr_v)
    pltpu.sync_copy(wc_h, wc_v)

    zvec = jnp.zeros((16,), jnp.float32)
    izero = jnp.zeros((16,), jnp.int32)
    for r in range(16):
        for q in range(D // 16):
            zb[r, pl.ds(q * 16, 16)] = zvec
    for s in range(2):
        didx[s][0, pl.ds(32, 16)] = izero
        sidx[s][0, pl.ds(32, 16)] = izero
        kbuf[s][0, pl.ds(32, 16)] = zvec - 1.0
        for r in range(MC, MCP):
            for q in range(D // 16):
                erows[s][r, pl.ds(q * 16, 16)] = zvec

    off = sid * _ZROWS
    nz = jnp.where(sid == 15, 40, 39)

    def zbody(i, carry):
        pltpu.sync_copy(zb, agg_sh.at[pl.ds(off + i * 16, 16)])
        return carry

    lax.fori_loop(0, nz, zbody, 0)
    plsc.subcore_barrier()

    lanes = jnp.arange(16, dtype=jnp.int32)

    def load_idx(c, s):
        base = tbase + c * MC
        pltpu.sync_copy(src_h.at[pl.ds(base, MC)], sidx[s].at[0, pl.ds(0, MC)])
        pltpu.sync_copy(dst_h.at[pl.ds(base, MC)], didx[s].at[0, pl.ds(0, MC)])
        pltpu.sync_copy(d2k_h.at[pl.ds(base, MC)], kbuf[s].at[0, pl.ds(0, MC)])

    def issue_gathers(c, s):
        base = tbase + c * MC
        pltpu.async_copy(a_h.at[sidx[s].at[0]], arows[s], gsem[s])
        pltpu.async_copy(b_h.at[didx[s].at[0]], brows[s], gsem[s])
        pltpu.async_copy(ea_h.at[pl.ds(base, MC)], erows[s].at[pl.ds(0, MC)], gsem[s])

    def wait_gathers(s):
        pltpu.make_async_copy(a_h.at[sidx[s].at[0]], arows[s], gsem[s]).wait()
        pltpu.make_async_copy(b_h.at[didx[s].at[0]], brows[s], gsem[s]).wait()
        pltpu.make_async_copy(ea_h.at[pl.ds(0, MC)], erows[s].at[pl.ds(0, MC)], gsem[s]).wait()

    def compute_chunk(c, s):
        keeps, d2s = [], []
        for g in range(MG):
            kv = kbuf[s][0, pl.ds(g * 16, 16)]
            keeps.append(jnp.where(kv >= 0.0, 1.0, 0.0))
            d2s.append(jnp.maximum(kv, 0.0))

        def jbody(j, dots):
            jv = lanes + j * 16
            w = plsc.load_gather(wr_v, [jv])
            wc = plsc.load_gather(wc_v, [jv])
            out = []
            for g in range(MG):
                ev = lanes + (g * 16)
                jcol = izero + j
                a = plsc.load_gather(arows[s], [ev, jcol])
                b = plsc.load_gather(brows[s], [ev, jcol])
                e = plsc.load_gather(erows[s], [ev, jcol])
                m = jnp.maximum(a + b + e + d2s[g] * w, 0.0) * keeps[g]
                plsc.store_scatter(arows[s], [ev, jcol], m)
                out.append(dots[g] + m * wc)
            return tuple(out)

        dots = lax.fori_loop(0, D, jbody,
                             tuple(jnp.zeros((16,), jnp.float32)
                                   for _ in range(MG)))
        for g in range(MG):
            tv = jnp.exp(dots[g] * 2.0)
            cbuf[s][0, pl.ds(g * 16, 16)] = (1.0 - 2.0 / (tv + 1.0)) * keeps[g]
        base = tbase + c * MC
        pltpu.sync_copy(cbuf[s].at[0, pl.ds(0, MC)], ct_h.at[pl.ds(base, MC)])
        pltpu.sync_copy(arows[s], agg_sh.at[didx[s].at[0]], add=True)

    load_idx(0, 0)
    issue_gathers(0, 0)
    load_idx(1, 1)
    last = MNCH - 1

    def pair_body(i, carry):
        c0 = 2 * i
        c1 = c0 + 1
        issue_gathers(c1, 1)
        wait_gathers(0)
        compute_chunk(c0, 0)
        load_idx(jnp.minimum(c0 + 2, last), 0)
        issue_gathers(jnp.minimum(c0 + 2, last), 0)
        wait_gathers(1)
        compute_chunk(c1, 1)
        load_idx(jnp.minimum(c1 + 2, last), 1)
        return carry

    lax.fori_loop(0, MNCH // 2, pair_body, 0)
    wait_gathers(0)
    plsc.subcore_barrier()

    @pl.when(sid < 15)
    def _():
        @pl.when(cid == 0)
        def _():
            pltpu.sync_copy(agg_sh.at[pl.ds(off, _ZROWS)], agg0_h.at[pl.ds(off, _ZROWS)])
        @pl.when(cid == 1)
        def _():
            pltpu.sync_copy(agg_sh.at[pl.ds(off, _ZROWS)], agg1_h.at[pl.ds(off, _ZROWS)])

    @pl.when(sid == 15)
    def _():
        @pl.when(cid == 0)
        def _():
            pltpu.sync_copy(agg_sh.at[pl.ds(off, 640)], agg0_h.at[pl.ds(off, 640)])
        @pl.when(cid == 1)
        def _():
            pltpu.sync_copy(agg_sh.at[pl.ds(off, 640)], agg1_h.at[pl.ds(off, 640)])


@functools.partial(
    pl.kernel,
    out_type=jax.ShapeDtypeStruct((NTILES, PW * N), jnp.float32),
    mesh=_MESH,
    scratch_types=[
        pltpu.VMEM((N,), jnp.float32),
        pltpu.VMEM((N,), jnp.float32),
        pltpu.VMEM((N,), jnp.float32),
        pltpu.VMEM((PW * N,), jnp.float32),
        pltpu.VMEM((SB,), jnp.int32),
        pltpu.VMEM((SB,), jnp.int32),
        pltpu.VMEM((SB,), jnp.float32),
        pltpu.VMEM((SB,), jnp.float32),
    ],
    compiler_params=_SCP)
def _sc_post(px_h, py_h, pz_h, src_h, dst_h, d2k_h, ct_h, pacc_h,
             px_v, py_v, pz_v, pacc_v, sbig, dbig, kbig, cbig):
    cid = lax.axis_index("c")
    sid = lax.axis_index("s")
    wid = cid * 16 + sid
    pltpu.sync_copy(px_h, px_v)
    pltpu.sync_copy(py_h, py_v)
    pltpu.sync_copy(pz_h, pz_v)

    zvec = jnp.zeros((16,), jnp.float32)

    def zb(i, carry):
        pacc_v[pl.ds(i * 16, 16)] = zvec
        return carry

    lax.fori_loop(0, PW * N // 16, zb, 0)

    def stage_body(t, carry):
        base = wid * TILE_E + t * SB
        pltpu.sync_copy(src_h.at[pl.ds(base, SB)], sbig)
        pltpu.sync_copy(dst_h.at[pl.ds(base, SB)], dbig)
        pltpu.sync_copy(d2k_h.at[pl.ds(base, SB)], kbig)
        pltpu.sync_copy(ct_h.at[pl.ds(base, SB)], cbig)

        def grp(g, carry2):
            sv = sbig[pl.ds(g * 16, 16)]
            dv = dbig[pl.ds(g * 16, 16)]
            kv = kbig[pl.ds(g * 16, 16)]
            ct = cbig[pl.ds(g * 16, 16)]
            kf = jnp.where(kv >= 0.0, 1.0, 0.0)
            dx = plsc.load_gather(px_v, [sv]) - plsc.load_gather(px_v, [dv])
            dy = plsc.load_gather(py_v, [sv]) - plsc.load_gather(py_v, [dv])
            dz = plsc.load_gather(pz_v, [sv]) - plsc.load_gather(pz_v, [dv])
            dj = dv * PW
            plsc.addupdate_scatter(pacc_v, [dj], dx * ct)
            plsc.addupdate_scatter(pacc_v, [dj + 1], dy * ct)
            plsc.addupdate_scatter(pacc_v, [dj + 2], dz * ct)
            plsc.addupdate_scatter(pacc_v, [dj + 3], kf)
            return carry2

        lax.fori_loop(0, SGRP, grp, 0)
        return carry

    lax.fori_loop(0, NSB, stage_body, 0)
    pltpu.sync_copy(pacc_v, pacc_h.at[wid])


def _edge_phase(visf_s, visf_d, a_t, b_t, ea_t, px, py, pz, src, dst,
                wr_flat, wc_flat):
    d2k = _sc_pre(visf_s, visf_d, px, py, pz, src, dst)
    agg0, agg1, ct = _sc_main(a_t, b_t, ea_t, d2k, src, dst, wr_flat, wc_flat)
    pacc_p = _sc_post(px, py, pz, src, dst, d2k, ct)
    pacc = _reduce_pacc(pacc_p.reshape(NTILES, PW * N))
    return agg0, agg1, pacc


def kernel(x, pos, edge_index, edge_attr, batch_indices, masked_token,
           enc_W_msg, enc_b_msg, enc_W_upd, enc_b_upd, enc_w_coord,
           dec_W_msg, dec_b_msg, dec_W_upd, dec_b_upd, dec_w_coord,
           dec_W_out, dec_b_out):
    del batch_indices
    src = edge_index[0]
    dst = edge_index[1]

    w1e, w2e, w3e = enc_W_msg[:D], enc_W_msg[D:2 * D], enc_W_msg[2 * D:2 * D + DE]
    w1d, w2d, w3d = dec_W_msg[:D], dec_W_msg[D:2 * D], dec_W_msg[2 * D:2 * D + DE]
    wr_e = jnp.tile(enc_W_msg[2 * D + DE][:, None], (1, 16)).reshape(-1)
    wr_d = jnp.tile(dec_W_msg[2 * D + DE][:, None], (1, 16)).reshape(-1)
    wc_e = jnp.tile(enc_w_coord, (1, 16)).reshape(-1)
    wc_d = jnp.tile(dec_w_coord, (1, 16)).reshape(-1)
    wu1e, wu2e = enc_W_upd[:D], enc_W_upd[D:]
    wu1d, wu2d = dec_W_upd[:D], dec_W_upd[D:]
    be = enc_b_msg[None, :]
    bd = dec_b_msg[None, :]
    bue = enc_b_upd[None, :]
    bud = dec_b_upd[None, :]

    pos8 = jnp.pad(pos, ((0, 0), (0, PW - 3)))
    mask_idx, visf, posm8, vis8 = _mask_constants()
    maskf = 1.0 - visf
    onesf = jnp.ones((N,), jnp.float32)

    a_t, b_t = _node_pre(x, w1e, w2e)
    ea_e, ea_d = _edge_pre(edge_attr, w3e, w3d, be, bd)

    agg0, agg1, pacc = _edge_phase(
        visf, visf, a_t, b_t, ea_e, pos[:, 0], pos[:, 1], pos[:, 2],
        src, dst, wr_e, wc_e)

    a2, b2, posc8 = _enc_upd(x, agg0, agg1, pacc, pos8, vis8,
                             posm8, masked_token, wu1e, wu2e, bue, w1d, w2d)

    agg20, agg21, pacc2 = _edge_phase(
        onesf, maskf, a2, b2, ea_d, posc8[:, 0], posc8[:, 1], posc8[:, 2],
        src, dst, wr_d, wc_d)

    wo8 = jnp.pad(dec_W_out, ((0, 0), (0, PW - 3)))
    bo8 = jnp.pad(dec_b_out, (0, PW - 3))[None, :]
    rec8 = _final(agg20, agg21, pacc2, masked_token,
                  wu1d, wu2d, bud, wo8, bo8, posm8)

    return rec8[mask_idx, :3], mask_idx


# speedup vs baseline: 1.5939x; 1.5939x over previous
import functools

import numpy as np
import jax
import jax.numpy as jnp
from jax import lax
from jax.experimental import pallas as pl
from jax.experimental.pallas import tpu as pltpu
from jax.experimental.pallas import tpu_sc as plsc

N = 10000
E = 320000
D = 128
DE = 16
PW = 8
NM = N // 2

NB = 5
BN = N // NB
EBG = 100
BEB = E // EBG

NTILES = 32
TILE_E = E // NTILES

MC = 40
MCP = 48
MNCH = TILE_E // MC
MG = MCP // 16

SB = 2000
NSB = TILE_E // SB
SGRP = SB // 16

_SCP = pltpu.CompilerParams(needs_layout_passes=False)
_MESH = plsc.VectorSubcoreMesh(core_axis_name="c", subcore_axis_name="s")


def _mask_constants():
    perm = jax.random.permutation(jax.random.key(42), N)
    mask_idx = perm[:NM]
    inv = jnp.argsort(perm)
    visf = (inv >= NM).astype(jnp.float32)
    pos_m = jax.random.normal(jax.random.key(7), (NM, 3), jnp.float32)
    pm_full = jnp.concatenate(
        [jnp.pad(pos_m, ((0, 0), (0, PW - 3))), jnp.zeros((NM, PW))], axis=0)
    posm8 = pm_full[inv]
    vis8 = jnp.pad(visf[:, None], ((0, 0), (0, PW - 1)))
    return mask_idx, visf, posm8, vis8


def _tc_node_pre(x_ref, w1_ref, w2_ref, a_ref, b_ref):
    xv = x_ref[...]
    a_ref[...] = jnp.dot(xv, w1_ref[...], preferred_element_type=jnp.float32)
    b_ref[...] = jnp.dot(xv, w2_ref[...], preferred_element_type=jnp.float32)


def _node_pre(x, w1, w2):
    return pl.pallas_call(
        _tc_node_pre,
        grid=(NB,),
        in_specs=[pl.BlockSpec((BN, D), lambda i: (i, 0)),
                  pl.BlockSpec((D, D), lambda i: (0, 0)),
                  pl.BlockSpec((D, D), lambda i: (0, 0))],
        out_specs=[pl.BlockSpec((BN, D), lambda i: (i, 0)),
                   pl.BlockSpec((BN, D), lambda i: (i, 0))],
        out_shape=[jax.ShapeDtypeStruct((N, D), jnp.float32),
                   jax.ShapeDtypeStruct((N, D), jnp.float32)],
    )(x, w1, w2)


def _tc_edge_pre(ea_ref, w3e_ref, w3d_ref, be_ref, bd_ref, oe_ref, od_ref):
    eav = ea_ref[...]
    oe_ref[...] = jnp.dot(eav, w3e_ref[...], preferred_element_type=jnp.float32) + be_ref[...]
    od_ref[...] = jnp.dot(eav, w3d_ref[...], preferred_element_type=jnp.float32) + bd_ref[...]


def _edge_pre(edge_attr, w3e, w3d, be, bd):
    return pl.pallas_call(
        _tc_edge_pre,
        grid=(EBG,),
        in_specs=[pl.BlockSpec((BEB, DE), lambda i: (i, 0)),
                  pl.BlockSpec((DE, D), lambda i: (0, 0)),
                  pl.BlockSpec((DE, D), lambda i: (0, 0)),
                  pl.BlockSpec((1, D), lambda i: (0, 0)),
                  pl.BlockSpec((1, D), lambda i: (0, 0))],
        out_specs=[pl.BlockSpec((BEB, D), lambda i: (i, 0)),
                   pl.BlockSpec((BEB, D), lambda i: (i, 0))],
        out_shape=[jax.ShapeDtypeStruct((E, D), jnp.float32),
                   jax.ShapeDtypeStruct((E, D), jnp.float32)],
    )(edge_attr, w3e, w3d, be, bd)


def _tc_reduce(p_ref, o_ref):
    o_ref[...] = jnp.sum(p_ref[...], axis=0, keepdims=True)


def _reduce_pacc(pacc_p):
    seg = PW * N // NB
    return pl.pallas_call(
        _tc_reduce,
        grid=(NB,),
        in_specs=[pl.BlockSpec((NTILES, seg), lambda i: (0, i))],
        out_specs=[pl.BlockSpec((1, seg), lambda i: (0, i))],
        out_shape=[jax.ShapeDtypeStruct((1, PW * N), jnp.float32)],
    )(pacc_p)[0].reshape(N, PW)


def _tc_enc_upd(x_ref, a0, a1, pacc_ref, pos_ref, vis_ref, posm_ref, tok_ref,
                wu1_ref, wu2_ref, bu_ref, w1d_ref, w2d_ref,
                a2_ref, b2_ref, pc_ref):
    agg = a0[...] + a1[...]
    pacc = pacc_ref[...]
    deg = pacc[:, 3:4] + 1.0
    h = jnp.maximum(
        jnp.dot(x_ref[...], wu1_ref[...], preferred_element_type=jnp.float32)
        + jnp.dot(agg / deg, wu2_ref[...], preferred_element_type=jnp.float32)
        + bu_ref[...], 0.0)
    vis = vis_ref[:, 0:1] > 0.0
    z = jnp.where(vis, h, tok_ref[...])
    pc_ref[...] = jnp.where(vis, pos_ref[...] + pacc / deg, posm_ref[...])
    a2_ref[...] = jnp.dot(z, w1d_ref[...], preferred_element_type=jnp.float32)
    b2_ref[...] = jnp.dot(z, w2d_ref[...], preferred_element_type=jnp.float32)


def _enc_upd(x, a0, a1, pacc, pos8, vis8, posm8, tok, wu1, wu2, bu, w1d, w2d):
    nd = lambda i: (i, 0)
    w0 = lambda i: (0, 0)
    return pl.pallas_call(
        _tc_enc_upd,
        grid=(NB,),
        in_specs=[pl.BlockSpec((BN, D), nd), pl.BlockSpec((BN, D), nd),
                  pl.BlockSpec((BN, D), nd), pl.BlockSpec((BN, PW), nd),
                  pl.BlockSpec((BN, PW), nd), pl.BlockSpec((BN, PW), nd),
                  pl.BlockSpec((BN, PW), nd),
                  pl.BlockSpec((1, D), w0),
                  pl.BlockSpec((D, D), w0), pl.BlockSpec((D, D), w0),
                  pl.BlockSpec((1, D), w0),
                  pl.BlockSpec((D, D), w0), pl.BlockSpec((D, D), w0)],
        out_specs=[pl.BlockSpec((BN, D), nd), pl.BlockSpec((BN, D), nd),
                   pl.BlockSpec((BN, PW), nd)],
        out_shape=[jax.ShapeDtypeStruct((N, D), jnp.float32),
                   jax.ShapeDtypeStruct((N, D), jnp.float32),
                   jax.ShapeDtypeStruct((N, PW), jnp.float32)],
    )(x, a0, a1, pacc, pos8, vis8, posm8, tok, wu1, wu2, bu, w1d, w2d)


def _tc_final(a0, a1, pacc_ref, tok_ref, wu1_ref, wu2_ref, bu_ref, wo_ref,
              bo_ref, posm_ref, out_ref):
    agg = a0[...] + a1[...]
    pacc = pacc_ref[...]
    deg = pacc[:, 3:4] + 1.0
    hz = jnp.maximum(
        jnp.dot(tok_ref[...], wu1_ref[...], preferred_element_type=jnp.float32)
        + jnp.dot(agg / deg, wu2_ref[...], preferred_element_type=jnp.float32)
        + bu_ref[...], 0.0)
    out_ref[...] = (jnp.dot(hz, wo_ref[...], preferred_element_type=jnp.float32)
                    + bo_ref[...] + posm_ref[...] + pacc / deg)


def _final(a0, a1, pacc, tok, wu1, wu2, bu, wo8, bo8, posm8):
    nd = lambda i: (i, 0)
    w0 = lambda i: (0, 0)
    return pl.pallas_call(
        _tc_final,
        grid=(NB,),
        in_specs=[pl.BlockSpec((BN, D), nd), pl.BlockSpec((BN, D), nd),
                  pl.BlockSpec((BN, PW), nd),
                  pl.BlockSpec((1, D), w0),
                  pl.BlockSpec((D, D), w0), pl.BlockSpec((D, D), w0),
                  pl.BlockSpec((1, D), w0),
                  pl.BlockSpec((D, PW), w0), pl.BlockSpec((1, PW), w0),
                  pl.BlockSpec((BN, PW), nd)],
        out_specs=[pl.BlockSpec((BN, PW), nd)],
        out_shape=[jax.ShapeDtypeStruct((N, PW), jnp.float32)],
    )(a0, a1, pacc, tok, wu1, wu2, bu, wo8, bo8, posm8)[0]


@functools.partial(
    pl.kernel,
    out_type=jax.ShapeDtypeStruct((E,), jnp.float32),
    mesh=_MESH,
    scratch_types=[
        pltpu.VMEM((N,), jnp.float32),
        pltpu.VMEM((N,), jnp.float32),
        pltpu.VMEM((N,), jnp.float32),
        pltpu.VMEM((N,), jnp.float32),
        pltpu.VMEM((N,), jnp.float32),
        pltpu.VMEM((SB,), jnp.int32),
        pltpu.VMEM((SB,), jnp.int32),
        pltpu.VMEM((SB,), jnp.float32),
    ],
    compiler_params=_SCP)
def _sc_pre(fs_h, fd_h, px_h, py_h, pz_h, src_h, dst_h, d2k_h,
            fs_v, fd_v, px_v, py_v, pz_v, sbig, dbig, obig):
    cid = lax.axis_index("c")
    sid = lax.axis_index("s")
    wid = cid * 16 + sid
    pltpu.sync_copy(fs_h, fs_v)
    pltpu.sync_copy(fd_h, fd_v)
    pltpu.sync_copy(px_h, px_v)
    pltpu.sync_copy(py_h, py_v)
    pltpu.sync_copy(pz_h, pz_v)

    def stage_body(t, carry):
        base = wid * TILE_E + t * SB
        pltpu.sync_copy(src_h.at[pl.ds(base, SB)], sbig)
        pltpu.sync_copy(dst_h.at[pl.ds(base, SB)], dbig)

        def grp(g, carry2):
            sv = sbig[pl.ds(g * 16, 16)]
            dv = dbig[pl.ds(g * 16, 16)]
            kf = plsc.load_gather(fs_v, [sv]) * plsc.load_gather(fd_v, [dv])
            dx = plsc.load_gather(px_v, [sv]) - plsc.load_gather(px_v, [dv])
            dy = plsc.load_gather(py_v, [sv]) - plsc.load_gather(py_v, [dv])
            dz = plsc.load_gather(pz_v, [sv]) - plsc.load_gather(pz_v, [dv])
            d2 = dx * dx + dy * dy + dz * dz
            obig[pl.ds(g * 16, 16)] = jnp.where(kf > 0.0, d2, -1.0)
            return carry2

        lax.fori_loop(0, SGRP, grp, 0)
        pltpu.sync_copy(obig, d2k_h.at[pl.ds(base, SB)])
        return carry

    lax.fori_loop(0, NSB, stage_body, 0)


_MAIN_OUT = [jax.ShapeDtypeStruct((N, D), jnp.float32),
             jax.ShapeDtypeStruct((N, D), jnp.float32),
             jax.ShapeDtypeStruct((E,), jnp.float32)]

_ZROWS = 624


@functools.partial(
    pl.kernel,
    out_type=_MAIN_OUT,
    mesh=_MESH,
    scratch_types=[
        pltpu.VMEM_SHARED((N, D), jnp.float32),
        [pltpu.VMEM((1, MCP), jnp.int32)] * 2,
        [pltpu.VMEM((1, MCP), jnp.int32)] * 2,
        [pltpu.VMEM((1, MCP + 16), jnp.float32)] * 2,
        [pltpu.VMEM((1, MCP), jnp.float32)] * 2,
        pltpu.VMEM((MCP,), jnp.float32),
        [pltpu.VMEM((MCP, D), jnp.float32)] * 2,
        [pltpu.VMEM((MCP, D), jnp.float32)] * 2,
        [pltpu.VMEM((MCP, D), jnp.float32)] * 2,
        pltpu.VMEM((D,), jnp.float32),
        pltpu.VMEM((D,), jnp.float32),
        pltpu.VMEM((16, D), jnp.float32),
        [pltpu.SemaphoreType.DMA] * 2,
    ],
    compiler_params=_SCP)
def _sc_main(a_h, b_h, ea_h, d2k_h, src_h, dst_h, wr_h, wc_h,
             agg0_h, agg1_h, ct_h,
             agg_sh, sidx, didx, kbuf, cbuf, dbuf, arows, brows, erows,
             wr_v, wc_v, zb, gsem):
    cid = lax.axis_index("c")
    sid = lax.axis_index("s")
    wid = cid * 16 + sid
    tbase = wid * TILE_E
    pltpu.sync_copy(wr_h, wr_v)
    pltpu.sync_copy(wc_h, wc_v)

    zvec = jnp.zeros((16,), jnp.float32)
    izero = jnp.zeros((16,), jnp.int32)
    for r in range(16):
        for q in range(D // 16):
            zb[r, pl.ds(q * 16, 16)] = zvec
    for s in range(2):
        didx[s][0, pl.ds(32, 16)] = izero
        sidx[s][0, pl.ds(32, 16)] = izero
        kbuf[s][0, pl.ds(32, 16)] = zvec - 1.0
        kbuf[s][0, pl.ds(48, 16)] = zvec - 1.0
        for r in range(MC, MCP):
            for q in range(D // 16):
                erows[s][r, pl.ds(q * 16, 16)] = zvec

    off = sid * _ZROWS
    nz = jnp.where(sid == 15, 40, 39)

    def zbody(i, carry):
        pltpu.sync_copy(zb, agg_sh.at[pl.ds(off + i * 16, 16)])
        return carry

    lax.fori_loop(0, nz, zbody, 0)
    plsc.subcore_barrier()

    lanes = jnp.arange(16, dtype=jnp.int32)

    def load_idx(c, s):
        base = tbase + c * MC
        pltpu.sync_copy(src_h.at[pl.ds(base, MC)], sidx[s].at[0, pl.ds(0, MC)])
        pltpu.sync_copy(dst_h.at[pl.ds(base, MC)], didx[s].at[0, pl.ds(0, MC)])
        pltpu.sync_copy(d2k_h.at[pl.ds(base, MC)], kbuf[s].at[0, pl.ds(0, MC)])

    def issue_gathers(c, s):
        base = tbase + c * MC
        pltpu.async_copy(a_h.at[sidx[s].at[0]], arows[s], gsem[s])
        pltpu.async_copy(b_h.at[didx[s].at[0]], brows[s], gsem[s])
        pltpu.async_copy(ea_h.at[pl.ds(base, MC)], erows[s].at[pl.ds(0, MC)], gsem[s])

    def wait_gathers(s):
        pltpu.make_async_copy(a_h.at[sidx[s].at[0]], arows[s], gsem[s]).wait()
        pltpu.make_async_copy(b_h.at[didx[s].at[0]], brows[s], gsem[s]).wait()
        pltpu.make_async_copy(ea_h.at[pl.ds(0, MC)], erows[s].at[pl.ds(0, MC)], gsem[s]).wait()

    wks = [wr_v[pl.ds(16 * k, 16)] for k in range(D // 16)]
    wcs = [wc_v[pl.ds(16 * k, 16)] for k in range(D // 16)]

    lane0 = lanes == 0

    def compute_chunk(c, s):
        def ebody(e, carry):
            kv = kbuf[s][0, pl.ds(e, 16)][0]
            keep = jnp.where(kv >= 0.0, 1.0, 0.0)
            d2 = jnp.maximum(kv, 0.0)
            dotv = jnp.zeros((16,), jnp.float32)
            for k in range(D // 16):
                a = arows[s][e, pl.ds(16 * k, 16)]
                b = brows[s][e, pl.ds(16 * k, 16)]
                ee = erows[s][e, pl.ds(16 * k, 16)]
                m = jnp.maximum(a + b + ee + d2 * wks[k], 0.0) * keep
                arows[s][e, pl.ds(16 * k, 16)] = m
                dotv = dotv + m * wcs[k]
            dot = jnp.sum(dotv)
            plsc.store_scatter(dbuf, [izero + e], jnp.zeros((16,), jnp.float32) + dot,
                               mask=lane0)
            return carry

        lax.fori_loop(0, MCP, ebody, 0)
        for g in range(MG):
            kv = kbuf[s][0, pl.ds(g * 16, 16)]
            keepv = jnp.where(kv >= 0.0, 1.0, 0.0)
            tv = jnp.exp(dbuf[pl.ds(g * 16, 16)] * 2.0)
            cbuf[s][0, pl.ds(g * 16, 16)] = (1.0 - 2.0 / (tv + 1.0)) * keepv
        base = tbase + c * MC
        pltpu.sync_copy(cbuf[s].at[0, pl.ds(0, MC)], ct_h.at[pl.ds(base, MC)])
        pltpu.sync_copy(arows[s], agg_sh.at[didx[s].at[0]], add=True)

    load_idx(0, 0)
    issue_gathers(0, 0)
    load_idx(1, 1)
    last = MNCH - 1

    def pair_body(i, carry):
        c0 = 2 * i
        c1 = c0 + 1
        issue_gathers(c1, 1)
        wait_gathers(0)
        compute_chunk(c0, 0)
        load_idx(jnp.minimum(c0 + 2, last), 0)
        issue_gathers(jnp.minimum(c0 + 2, last), 0)
        wait_gathers(1)
        compute_chunk(c1, 1)
        load_idx(jnp.minimum(c1 + 2, last), 1)
        return carry

    lax.fori_loop(0, MNCH // 2, pair_body, 0)
    wait_gathers(0)
    plsc.subcore_barrier()

    @pl.when(sid < 15)
    def _():
        @pl.when(cid == 0)
        def _():
            pltpu.sync_copy(agg_sh.at[pl.ds(off, _ZROWS)], agg0_h.at[pl.ds(off, _ZROWS)])
        @pl.when(cid == 1)
        def _():
            pltpu.sync_copy(agg_sh.at[pl.ds(off, _ZROWS)], agg1_h.at[pl.ds(off, _ZROWS)])

    @pl.when(sid == 15)
    def _():
        @pl.when(cid == 0)
        def _():
            pltpu.sync_copy(agg_sh.at[pl.ds(off, 640)], agg0_h.at[pl.ds(off, 640)])
        @pl.when(cid == 1)
        def _():
            pltpu.sync_copy(agg_sh.at[pl.ds(off, 640)], agg1_h.at[pl.ds(off, 640)])


@functools.partial(
    pl.kernel,
    out_type=jax.ShapeDtypeStruct((NTILES, PW * N), jnp.float32),
    mesh=_MESH,
    scratch_types=[
        pltpu.VMEM((N,), jnp.float32),
        pltpu.VMEM((N,), jnp.float32),
        pltpu.VMEM((N,), jnp.float32),
        pltpu.VMEM((PW * N,), jnp.float32),
        pltpu.VMEM((SB,), jnp.int32),
        pltpu.VMEM((SB,), jnp.int32),
        pltpu.VMEM((SB,), jnp.float32),
        pltpu.VMEM((SB,), jnp.float32),
    ],
    compiler_params=_SCP)
def _sc_post(px_h, py_h, pz_h, src_h, dst_h, d2k_h, ct_h, pacc_h,
             px_v, py_v, pz_v, pacc_v, sbig, dbig, kbig, cbig):
    cid = lax.axis_index("c")
    sid = lax.axis_index("s")
    wid = cid * 16 + sid
    pltpu.sync_copy(px_h, px_v)
    pltpu.sync_copy(py_h, py_v)
    pltpu.sync_copy(pz_h, pz_v)

    zvec = jnp.zeros((16,), jnp.float32)

    def zb(i, carry):
        pacc_v[pl.ds(i * 16, 16)] = zvec
        return carry

    lax.fori_loop(0, PW * N // 16, zb, 0)

    def stage_body(t, carry):
        base = wid * TILE_E + t * SB
        pltpu.sync_copy(src_h.at[pl.ds(base, SB)], sbig)
        pltpu.sync_copy(dst_h.at[pl.ds(base, SB)], dbig)
        pltpu.sync_copy(d2k_h.at[pl.ds(base, SB)], kbig)
        pltpu.sync_copy(ct_h.at[pl.ds(base, SB)], cbig)

        def grp(g, carry2):
            sv = sbig[pl.ds(g * 16, 16)]
            dv = dbig[pl.ds(g * 16, 16)]
            kv = kbig[pl.ds(g * 16, 16)]
            ct = cbig[pl.ds(g * 16, 16)]
            kf = jnp.where(kv >= 0.0, 1.0, 0.0)
            dx = plsc.load_gather(px_v, [sv]) - plsc.load_gather(px_v, [dv])
            dy = plsc.load_gather(py_v, [sv]) - plsc.load_gather(py_v, [dv])
            dz = plsc.load_gather(pz_v, [sv]) - plsc.load_gather(pz_v, [dv])
            dj = dv * PW
            plsc.addupdate_scatter(pacc_v, [dj], dx * ct)
            plsc.addupdate_scatter(pacc_v, [dj + 1], dy * ct)
            plsc.addupdate_scatter(pacc_v, [dj + 2], dz * ct)
            plsc.addupdate_scatter(pacc_v, [dj + 3], kf)
            return carry2

        lax.fori_loop(0, SGRP, grp, 0)
        return carry

    lax.fori_loop(0, NSB, stage_body, 0)
    pltpu.sync_copy(pacc_v, pacc_h.at[wid])


def _edge_phase(visf_s, visf_d, a_t, b_t, ea_t, px, py, pz, src, dst,
                wr_flat, wc_flat):
    d2k = _sc_pre(visf_s, visf_d, px, py, pz, src, dst)
    agg0, agg1, ct = _sc_main(a_t, b_t, ea_t, d2k, src, dst, wr_flat, wc_flat)
    pacc_p = _sc_post(px, py, pz, src, dst, d2k, ct)
    pacc = _reduce_pacc(pacc_p.reshape(NTILES, PW * N))
    return agg0, agg1, pacc


def kernel(x, pos, edge_index, edge_attr, batch_indices, masked_token,
           enc_W_msg, enc_b_msg, enc_W_upd, enc_b_upd, enc_w_coord,
           dec_W_msg, dec_b_msg, dec_W_upd, dec_b_upd, dec_w_coord,
           dec_W_out, dec_b_out):
    del batch_indices
    src = edge_index[0]
    dst = edge_index[1]

    w1e, w2e, w3e = enc_W_msg[:D], enc_W_msg[D:2 * D], enc_W_msg[2 * D:2 * D + DE]
    w1d, w2d, w3d = dec_W_msg[:D], dec_W_msg[D:2 * D], dec_W_msg[2 * D:2 * D + DE]
    wr_e = enc_W_msg[2 * D + DE]
    wr_d = dec_W_msg[2 * D + DE]
    wc_e = enc_w_coord[:, 0]
    wc_d = dec_w_coord[:, 0]
    wu1e, wu2e = enc_W_upd[:D], enc_W_upd[D:]
    wu1d, wu2d = dec_W_upd[:D], dec_W_upd[D:]
    be = enc_b_msg[None, :]
    bd = dec_b_msg[None, :]
    bue = enc_b_upd[None, :]
    bud = dec_b_upd[None, :]

    pos8 = jnp.pad(pos, ((0, 0), (0, PW - 3)))
    mask_idx, visf, posm8, vis8 = _mask_constants()
    maskf = 1.0 - visf
    onesf = jnp.ones((N,), jnp.float32)

    a_t, b_t = _node_pre(x, w1e, w2e)
    ea_e, ea_d = _edge_pre(edge_attr, w3e, w3d, be, bd)

    agg0, agg1, pacc = _edge_phase(
        visf, visf, a_t, b_t, ea_e, pos[:, 0], pos[:, 1], pos[:, 2],
        src, dst, wr_e, wc_e)

    a2, b2, posc8 = _enc_upd(x, agg0, agg1, pacc, pos8, vis8,
                             posm8, masked_token, wu1e, wu2e, bue, w1d, w2d)

    agg20, agg21, pacc2 = _edge_phase(
        onesf, maskf, a2, b2, ea_d, posc8[:, 0], posc8[:, 1], posc8[:, 2],
        src, dst, wr_d, wc_d)

    wo8 = jnp.pad(dec_W_out, ((0, 0), (0, PW - 3)))
    bo8 = jnp.pad(dec_b_out, (0, PW - 3))[None, :]
    rec8 = _final(agg20, agg21, pacc2, masked_token,
                  wu1d, wu2d, bud, wo8, bo8, posm8)

    return rec8[mask_idx, :3], mask_idx


# speedup vs baseline: 1.6008x; 1.0044x over previous
import functools

import numpy as np
import jax
import jax.numpy as jnp
from jax import lax
from jax.experimental import pallas as pl
from jax.experimental.pallas import tpu as pltpu
from jax.experimental.pallas import tpu_sc as plsc

N = 10000
E = 320000
D = 128
DE = 16
PW = 8
NM = N // 2

NB = 5
BN = N // NB
EBG = 100
BEB = E // EBG

NTILES = 32
TILE_E = E // NTILES

MC = 40
MCP = 48
MNCH = TILE_E // MC
MG = MCP // 16

SB = 2000
NSB = TILE_E // SB
SGRP = SB // 16

_SCP = pltpu.CompilerParams(needs_layout_passes=False)
_MESH = plsc.VectorSubcoreMesh(core_axis_name="c", subcore_axis_name="s")


def _mask_constants():
    perm = jax.random.permutation(jax.random.key(42), N)
    mask_idx = perm[:NM]
    inv = jnp.argsort(perm)
    visf = (inv >= NM).astype(jnp.float32)
    pos_m = jax.random.normal(jax.random.key(7), (NM, 3), jnp.float32)
    pm_full = jnp.concatenate(
        [jnp.pad(pos_m, ((0, 0), (0, PW - 3))), jnp.zeros((NM, PW))], axis=0)
    posm8 = pm_full[inv]
    vis8 = jnp.pad(visf[:, None], ((0, 0), (0, PW - 1)))
    return mask_idx, visf, posm8, vis8


def _tc_node_pre(x_ref, w1_ref, w2_ref, a_ref, b_ref):
    xv = x_ref[...]
    a_ref[...] = jnp.dot(xv, w1_ref[...], preferred_element_type=jnp.float32)
    b_ref[...] = jnp.dot(xv, w2_ref[...], preferred_element_type=jnp.float32)


def _node_pre(x, w1, w2):
    return pl.pallas_call(
        _tc_node_pre,
        grid=(NB,),
        in_specs=[pl.BlockSpec((BN, D), lambda i: (i, 0)),
                  pl.BlockSpec((D, D), lambda i: (0, 0)),
                  pl.BlockSpec((D, D), lambda i: (0, 0))],
        out_specs=[pl.BlockSpec((BN, D), lambda i: (i, 0)),
                   pl.BlockSpec((BN, D), lambda i: (i, 0))],
        out_shape=[jax.ShapeDtypeStruct((N, D), jnp.float32),
                   jax.ShapeDtypeStruct((N, D), jnp.float32)],
    )(x, w1, w2)


def _tc_edge_pre(ea_ref, w3e_ref, w3d_ref, be_ref, bd_ref, oe_ref, od_ref):
    eav = ea_ref[...]
    oe_ref[...] = jnp.dot(eav, w3e_ref[...], preferred_element_type=jnp.float32) + be_ref[...]
    od_ref[...] = jnp.dot(eav, w3d_ref[...], preferred_element_type=jnp.float32) + bd_ref[...]


def _edge_pre(edge_attr, w3e, w3d, be, bd):
    return pl.pallas_call(
        _tc_edge_pre,
        grid=(EBG,),
        in_specs=[pl.BlockSpec((BEB, DE), lambda i: (i, 0)),
                  pl.BlockSpec((DE, D), lambda i: (0, 0)),
                  pl.BlockSpec((DE, D), lambda i: (0, 0)),
                  pl.BlockSpec((1, D), lambda i: (0, 0)),
                  pl.BlockSpec((1, D), lambda i: (0, 0))],
        out_specs=[pl.BlockSpec((BEB, D), lambda i: (i, 0)),
                   pl.BlockSpec((BEB, D), lambda i: (i, 0))],
        out_shape=[jax.ShapeDtypeStruct((E, D), jnp.float32),
                   jax.ShapeDtypeStruct((E, D), jnp.float32)],
    )(edge_attr, w3e, w3d, be, bd)


def _tc_reduce(p_ref, o_ref):
    o_ref[...] = jnp.sum(p_ref[...], axis=0, keepdims=True)


def _reduce_pacc(pacc_p):
    seg = PW * N // NB
    return pl.pallas_call(
        _tc_reduce,
        grid=(NB,),
        in_specs=[pl.BlockSpec((NTILES, seg), lambda i: (0, i))],
        out_specs=[pl.BlockSpec((1, seg), lambda i: (0, i))],
        out_shape=[jax.ShapeDtypeStruct((1, PW * N), jnp.float32)],
    )(pacc_p)[0].reshape(N, PW)


def _tc_enc_upd(x_ref, a0, a1, pacc_ref, pos_ref, vis_ref, posm_ref, tok_ref,
                wu1_ref, wu2_ref, bu_ref, w1d_ref, w2d_ref,
                a2_ref, b2_ref, pc_ref):
    agg = a0[...] + a1[...]
    pacc = pacc_ref[...]
    deg = pacc[:, 3:4] + 1.0
    h = jnp.maximum(
        jnp.dot(x_ref[...], wu1_ref[...], preferred_element_type=jnp.float32)
        + jnp.dot(agg / deg, wu2_ref[...], preferred_element_type=jnp.float32)
        + bu_ref[...], 0.0)
    vis = vis_ref[:, 0:1] > 0.0
    z = jnp.where(vis, h, tok_ref[...])
    pc_ref[...] = jnp.where(vis, pos_ref[...] + pacc / deg, posm_ref[...])
    a2_ref[...] = jnp.dot(z, w1d_ref[...], preferred_element_type=jnp.float32)
    b2_ref[...] = jnp.dot(z, w2d_ref[...], preferred_element_type=jnp.float32)


def _enc_upd(x, a0, a1, pacc, pos8, vis8, posm8, tok, wu1, wu2, bu, w1d, w2d):
    nd = lambda i: (i, 0)
    w0 = lambda i: (0, 0)
    return pl.pallas_call(
        _tc_enc_upd,
        grid=(NB,),
        in_specs=[pl.BlockSpec((BN, D), nd), pl.BlockSpec((BN, D), nd),
                  pl.BlockSpec((BN, D), nd), pl.BlockSpec((BN, PW), nd),
                  pl.BlockSpec((BN, PW), nd), pl.BlockSpec((BN, PW), nd),
                  pl.BlockSpec((BN, PW), nd),
                  pl.BlockSpec((1, D), w0),
                  pl.BlockSpec((D, D), w0), pl.BlockSpec((D, D), w0),
                  pl.BlockSpec((1, D), w0),
                  pl.BlockSpec((D, D), w0), pl.BlockSpec((D, D), w0)],
        out_specs=[pl.BlockSpec((BN, D), nd), pl.BlockSpec((BN, D), nd),
                   pl.BlockSpec((BN, PW), nd)],
        out_shape=[jax.ShapeDtypeStruct((N, D), jnp.float32),
                   jax.ShapeDtypeStruct((N, D), jnp.float32),
                   jax.ShapeDtypeStruct((N, PW), jnp.float32)],
    )(x, a0, a1, pacc, pos8, vis8, posm8, tok, wu1, wu2, bu, w1d, w2d)


def _tc_final(a0, a1, pacc_ref, tok_ref, wu1_ref, wu2_ref, bu_ref, wo_ref,
              bo_ref, posm_ref, out_ref):
    agg = a0[...] + a1[...]
    pacc = pacc_ref[...]
    deg = pacc[:, 3:4] + 1.0
    hz = jnp.maximum(
        jnp.dot(tok_ref[...], wu1_ref[...], preferred_element_type=jnp.float32)
        + jnp.dot(agg / deg, wu2_ref[...], preferred_element_type=jnp.float32)
        + bu_ref[...], 0.0)
    out_ref[...] = (jnp.dot(hz, wo_ref[...], preferred_element_type=jnp.float32)
                    + bo_ref[...] + posm_ref[...] + pacc / deg)


def _final(a0, a1, pacc, tok, wu1, wu2, bu, wo8, bo8, posm8):
    nd = lambda i: (i, 0)
    w0 = lambda i: (0, 0)
    return pl.pallas_call(
        _tc_final,
        grid=(NB,),
        in_specs=[pl.BlockSpec((BN, D), nd), pl.BlockSpec((BN, D), nd),
                  pl.BlockSpec((BN, PW), nd),
                  pl.BlockSpec((1, D), w0),
                  pl.BlockSpec((D, D), w0), pl.BlockSpec((D, D), w0),
                  pl.BlockSpec((1, D), w0),
                  pl.BlockSpec((D, PW), w0), pl.BlockSpec((1, PW), w0),
                  pl.BlockSpec((BN, PW), nd)],
        out_specs=[pl.BlockSpec((BN, PW), nd)],
        out_shape=[jax.ShapeDtypeStruct((N, PW), jnp.float32)],
    )(a0, a1, pacc, tok, wu1, wu2, bu, wo8, bo8, posm8)[0]


@functools.partial(
    pl.kernel,
    out_type=jax.ShapeDtypeStruct((E,), jnp.float32),
    mesh=_MESH,
    scratch_types=[
        pltpu.VMEM((N,), jnp.float32),
        pltpu.VMEM((N,), jnp.float32),
        pltpu.VMEM((N,), jnp.float32),
        pltpu.VMEM((N,), jnp.float32),
        pltpu.VMEM((N,), jnp.float32),
        pltpu.VMEM((SB,), jnp.int32),
        pltpu.VMEM((SB,), jnp.int32),
        pltpu.VMEM((SB,), jnp.float32),
    ],
    compiler_params=_SCP)
def _sc_pre(fs_h, fd_h, px_h, py_h, pz_h, src_h, dst_h, d2k_h,
            fs_v, fd_v, px_v, py_v, pz_v, sbig, dbig, obig):
    cid = lax.axis_index("c")
    sid = lax.axis_index("s")
    wid = cid * 16 + sid
    pltpu.sync_copy(fs_h, fs_v)
    pltpu.sync_copy(fd_h, fd_v)
    pltpu.sync_copy(px_h, px_v)
    pltpu.sync_copy(py_h, py_v)
    pltpu.sync_copy(pz_h, pz_v)

    def stage_body(t, carry):
        base = wid * TILE_E + t * SB
        pltpu.sync_copy(src_h.at[pl.ds(base, SB)], sbig)
        pltpu.sync_copy(dst_h.at[pl.ds(base, SB)], dbig)

        def grp(g, carry2):
            sv = sbig[pl.ds(g * 16, 16)]
            dv = dbig[pl.ds(g * 16, 16)]
            kf = plsc.load_gather(fs_v, [sv]) * plsc.load_gather(fd_v, [dv])
            dx = plsc.load_gather(px_v, [sv]) - plsc.load_gather(px_v, [dv])
            dy = plsc.load_gather(py_v, [sv]) - plsc.load_gather(py_v, [dv])
            dz = plsc.load_gather(pz_v, [sv]) - plsc.load_gather(pz_v, [dv])
            d2 = dx * dx + dy * dy + dz * dz
            obig[pl.ds(g * 16, 16)] = jnp.where(kf > 0.0, d2, -1.0)
            return carry2

        lax.fori_loop(0, SGRP, grp, 0)
        pltpu.sync_copy(obig, d2k_h.at[pl.ds(base, SB)])
        return carry

    lax.fori_loop(0, NSB, stage_body, 0)


_MAIN_OUT = [jax.ShapeDtypeStruct((N, D), jnp.float32),
             jax.ShapeDtypeStruct((N, D), jnp.float32),
             jax.ShapeDtypeStruct((E,), jnp.float32)]

_ZROWS = 624


@functools.partial(
    pl.kernel,
    out_type=_MAIN_OUT,
    mesh=_MESH,
    scratch_types=[
        pltpu.VMEM_SHARED((N, D), jnp.float32),
        [pltpu.VMEM((1, MCP), jnp.int32)] * 2,
        [pltpu.VMEM((1, MCP), jnp.int32)] * 2,
        [pltpu.VMEM((1, MCP + 16), jnp.float32)] * 2,
        [pltpu.VMEM((1, MCP), jnp.float32)] * 2,
        pltpu.VMEM((MCP,), jnp.float32),
        [pltpu.VMEM((MCP, D), jnp.float32)] * 2,
        [pltpu.VMEM((MCP, D), jnp.float32)] * 2,
        [pltpu.VMEM((MCP, D), jnp.float32)] * 2,
        pltpu.VMEM((D,), jnp.float32),
        pltpu.VMEM((D,), jnp.float32),
        pltpu.VMEM((16, D), jnp.float32),
        [pltpu.SemaphoreType.DMA] * 2,
    ],
    compiler_params=_SCP)
def _sc_main(a_h, b_h, ea_h, d2k_h, src_h, dst_h, wr_h, wc_h,
             agg0_h, agg1_h, ct_h,
             agg_sh, sidx, didx, kbuf, cbuf, dbuf, arows, brows, erows,
             wr_v, wc_v, zb, gsem):
    cid = lax.axis_index("c")
    sid = lax.axis_index("s")
    wid = cid * 16 + sid
    tbase = wid * TILE_E
    pltpu.sync_copy(wr_h, wr_v)
    pltpu.sync_copy(wc_h, wc_v)

    zvec = jnp.zeros((16,), jnp.float32)
    izero = jnp.zeros((16,), jnp.int32)
    for r in range(16):
        for q in range(D // 16):
            zb[r, pl.ds(q * 16, 16)] = zvec
    for s in range(2):
        didx[s][0, pl.ds(32, 16)] = izero
        sidx[s][0, pl.ds(32, 16)] = izero
        kbuf[s][0, pl.ds(32, 16)] = zvec - 1.0
        kbuf[s][0, pl.ds(48, 16)] = zvec - 1.0
        for r in range(MC, MCP):
            for q in range(D // 16):
                erows[s][r, pl.ds(q * 16, 16)] = zvec

    off = sid * _ZROWS
    nz = jnp.where(sid == 15, 40, 39)

    def zbody(i, carry):
        pltpu.sync_copy(zb, agg_sh.at[pl.ds(off + i * 16, 16)])
        return carry

    lax.fori_loop(0, nz, zbody, 0)
    plsc.subcore_barrier()

    lanes = jnp.arange(16, dtype=jnp.int32)

    def load_idx(c, s):
        base = tbase + c * MC
        pltpu.sync_copy(src_h.at[pl.ds(base, MC)], sidx[s].at[0, pl.ds(0, MC)])
        pltpu.sync_copy(dst_h.at[pl.ds(base, MC)], didx[s].at[0, pl.ds(0, MC)])
        pltpu.sync_copy(d2k_h.at[pl.ds(base, MC)], kbuf[s].at[0, pl.ds(0, MC)])

    def issue_gathers(c, s):
        base = tbase + c * MC
        pltpu.async_copy(a_h.at[sidx[s].at[0]], arows[s], gsem[s])
        pltpu.async_copy(b_h.at[didx[s].at[0]], brows[s], gsem[s])
        pltpu.async_copy(ea_h.at[pl.ds(base, MC)], erows[s].at[pl.ds(0, MC)], gsem[s])

    def wait_gathers(s):
        pltpu.make_async_copy(a_h.at[sidx[s].at[0]], arows[s], gsem[s]).wait()
        pltpu.make_async_copy(b_h.at[didx[s].at[0]], brows[s], gsem[s]).wait()
        pltpu.make_async_copy(ea_h.at[pl.ds(0, MC)], erows[s].at[pl.ds(0, MC)], gsem[s]).wait()

    wks = [wr_v[pl.ds(16 * k, 16)] for k in range(D // 16)]
    wcs = [wc_v[pl.ds(16 * k, 16)] for k in range(D // 16)]

    lane0 = lanes == 0

    def compute_chunk(c, s):
        @plsc.parallel_loop(0, MCP, unroll=4)
        def ebody(e):
            kv = kbuf[s][0, pl.ds(e, 16)][0]
            keep = jnp.where(kv >= 0.0, 1.0, 0.0)
            d2 = jnp.maximum(kv, 0.0)
            dotv = jnp.zeros((16,), jnp.float32)
            for k in range(D // 16):
                a = arows[s][e, pl.ds(16 * k, 16)]
                b = brows[s][e, pl.ds(16 * k, 16)]
                ee = erows[s][e, pl.ds(16 * k, 16)]
                m = jnp.maximum(a + b + ee + d2 * wks[k], 0.0) * keep
                arows[s][e, pl.ds(16 * k, 16)] = m
                dotv = dotv + m * wcs[k]
            dot = jnp.sum(dotv)
            plsc.store_scatter(dbuf, [izero + e], jnp.zeros((16,), jnp.float32) + dot,
                               mask=lane0)
        for g in range(MG):
            kv = kbuf[s][0, pl.ds(g * 16, 16)]
            keepv = jnp.where(kv >= 0.0, 1.0, 0.0)
            tv = jnp.exp(dbuf[pl.ds(g * 16, 16)] * 2.0)
            cbuf[s][0, pl.ds(g * 16, 16)] = (1.0 - 2.0 / (tv + 1.0)) * keepv
        base = tbase + c * MC
        pltpu.sync_copy(cbuf[s].at[0, pl.ds(0, MC)], ct_h.at[pl.ds(base, MC)])
        pltpu.sync_copy(arows[s], agg_sh.at[didx[s].at[0]], add=True)

    load_idx(0, 0)
    issue_gathers(0, 0)
    load_idx(1, 1)
    last = MNCH - 1

    def pair_body(i, carry):
        c0 = 2 * i
        c1 = c0 + 1
        issue_gathers(c1, 1)
        wait_gathers(0)
        compute_chunk(c0, 0)
        load_idx(jnp.minimum(c0 + 2, last), 0)
        issue_gathers(jnp.minimum(c0 + 2, last), 0)
        wait_gathers(1)
        compute_chunk(c1, 1)
        load_idx(jnp.minimum(c1 + 2, last), 1)
        return carry

    lax.fori_loop(0, MNCH // 2, pair_body, 0)
    wait_gathers(0)
    plsc.subcore_barrier()

    @pl.when(sid < 15)
    def _():
        @pl.when(cid == 0)
        def _():
            pltpu.sync_copy(agg_sh.at[pl.ds(off, _ZROWS)], agg0_h.at[pl.ds(off, _ZROWS)])
        @pl.when(cid == 1)
        def _():
            pltpu.sync_copy(agg_sh.at[pl.ds(off, _ZROWS)], agg1_h.at[pl.ds(off, _ZROWS)])

    @pl.when(sid == 15)
    def _():
        @pl.when(cid == 0)
        def _():
            pltpu.sync_copy(agg_sh.at[pl.ds(off, 640)], agg0_h.at[pl.ds(off, 640)])
        @pl.when(cid == 1)
        def _():
            pltpu.sync_copy(agg_sh.at[pl.ds(off, 640)], agg1_h.at[pl.ds(off, 640)])


@functools.partial(
    pl.kernel,
    out_type=jax.ShapeDtypeStruct((NTILES, PW * N), jnp.float32),
    mesh=_MESH,
    scratch_types=[
        pltpu.VMEM((N,), jnp.float32),
        pltpu.VMEM((N,), jnp.float32),
        pltpu.VMEM((N,), jnp.float32),
        pltpu.VMEM((PW * N,), jnp.float32),
        pltpu.VMEM((SB,), jnp.int32),
        pltpu.VMEM((SB,), jnp.int32),
        pltpu.VMEM((SB,), jnp.float32),
        pltpu.VMEM((SB,), jnp.float32),
    ],
    compiler_params=_SCP)
def _sc_post(px_h, py_h, pz_h, src_h, dst_h, d2k_h, ct_h, pacc_h,
             px_v, py_v, pz_v, pacc_v, sbig, dbig, kbig, cbig):
    cid = lax.axis_index("c")
    sid = lax.axis_index("s")
    wid = cid * 16 + sid
    pltpu.sync_copy(px_h, px_v)
    pltpu.sync_copy(py_h, py_v)
    pltpu.sync_copy(pz_h, pz_v)

    zvec = jnp.zeros((16,), jnp.float32)

    def zb(i, carry):
        pacc_v[pl.ds(i * 16, 16)] = zvec
        return carry

    lax.fori_loop(0, PW * N // 16, zb, 0)

    def stage_body(t, carry):
        base = wid * TILE_E + t * SB
        pltpu.sync_copy(src_h.at[pl.ds(base, SB)], sbig)
        pltpu.sync_copy(dst_h.at[pl.ds(base, SB)], dbig)
        pltpu.sync_copy(d2k_h.at[pl.ds(base, SB)], kbig)
        pltpu.sync_copy(ct_h.at[pl.ds(base, SB)], cbig)

        def grp(g, carry2):
            sv = sbig[pl.ds(g * 16, 16)]
            dv = dbig[pl.ds(g * 16, 16)]
            kv = kbig[pl.ds(g * 16, 16)]
            ct = cbig[pl.ds(g * 16, 16)]
            kf = jnp.where(kv >= 0.0, 1.0, 0.0)
            dx = plsc.load_gather(px_v, [sv]) - plsc.load_gather(px_v, [dv])
            dy = plsc.load_gather(py_v, [sv]) - plsc.load_gather(py_v, [dv])
            dz = plsc.load_gather(pz_v, [sv]) - plsc.load_gather(pz_v, [dv])
            dj = dv * PW
            plsc.addupdate_scatter(pacc_v, [dj], dx * ct)
            plsc.addupdate_scatter(pacc_v, [dj + 1], dy * ct)
            plsc.addupdate_scatter(pacc_v, [dj + 2], dz * ct)
            plsc.addupdate_scatter(pacc_v, [dj + 3], kf)
            return carry2

        lax.fori_loop(0, SGRP, grp, 0)
        return carry

    lax.fori_loop(0, NSB, stage_body, 0)
    pltpu.sync_copy(pacc_v, pacc_h.at[wid])


def _edge_phase(visf_s, visf_d, a_t, b_t, ea_t, px, py, pz, src, dst,
                wr_flat, wc_flat):
    d2k = _sc_pre(visf_s, visf_d, px, py, pz, src, dst)
    agg0, agg1, ct = _sc_main(a_t, b_t, ea_t, d2k, src, dst, wr_flat, wc_flat)
    pacc_p = _sc_post(px, py, pz, src, dst, d2k, ct)
    pacc = _reduce_pacc(pacc_p.reshape(NTILES, PW * N))
    return agg0, agg1, pacc


def kernel(x, pos, edge_index, edge_attr, batch_indices, masked_token,
           enc_W_msg, enc_b_msg, enc_W_upd, enc_b_upd, enc_w_coord,
           dec_W_msg, dec_b_msg, dec_W_upd, dec_b_upd, dec_w_coord,
           dec_W_out, dec_b_out):
    del batch_indices
    src = edge_index[0]
    dst = edge_index[1]

    w1e, w2e, w3e = enc_W_msg[:D], enc_W_msg[D:2 * D], enc_W_msg[2 * D:2 * D + DE]
    w1d, w2d, w3d = dec_W_msg[:D], dec_W_msg[D:2 * D], dec_W_msg[2 * D:2 * D + DE]
    wr_e = enc_W_msg[2 * D + DE]
    wr_d = dec_W_msg[2 * D + DE]
    wc_e = enc_w_coord[:, 0]
    wc_d = dec_w_coord[:, 0]
    wu1e, wu2e = enc_W_upd[:D], enc_W_upd[D:]
    wu1d, wu2d = dec_W_upd[:D], dec_W_upd[D:]
    be = enc_b_msg[None, :]
    bd = dec_b_msg[None, :]
    bue = enc_b_upd[None, :]
    bud = dec_b_upd[None, :]

    pos8 = jnp.pad(pos, ((0, 0), (0, PW - 3)))
    mask_idx, visf, posm8, vis8 = _mask_constants()
    maskf = 1.0 - visf
    onesf = jnp.ones((N,), jnp.float32)

    a_t, b_t = _node_pre(x, w1e, w2e)
    ea_e, ea_d = _edge_pre(edge_attr, w3e, w3d, be, bd)

    agg0, agg1, pacc = _edge_phase(
        visf, visf, a_t, b_t, ea_e, pos[:, 0], pos[:, 1], pos[:, 2],
        src, dst, wr_e, wc_e)

    a2, b2, posc8 = _enc_upd(x, agg0, agg1, pacc, pos8, vis8,
                             posm8, masked_token, wu1e, wu2e, bue, w1d, w2d)

    agg20, agg21, pacc2 = _edge_phase(
        onesf, maskf, a2, b2, ea_d, posc8[:, 0], posc8[:, 1], posc8[:, 2],
        src, dst, wr_d, wc_d)

    wo8 = jnp.pad(dec_W_out, ((0, 0), (0, PW - 3)))
    bo8 = jnp.pad(dec_b_out, (0, PW - 3))[None, :]
    rec8 = _final(agg20, agg21, pacc2, masked_token,
                  wu1d, wu2d, bud, wo8, bo8, posm8)

    return rec8[mask_idx, :3], mask_idx


# speedup vs baseline: 1.6047x; 1.0024x over previous
import functools

import numpy as np
import jax
import jax.numpy as jnp
from jax import lax
from jax.experimental import pallas as pl
from jax.experimental.pallas import tpu as pltpu
from jax.experimental.pallas import tpu_sc as plsc

N = 10000
E = 320000
D = 128
DE = 16
PW = 8
NM = N // 2

NB = 5
BN = N // NB
EBG = 100
BEB = E // EBG

NTILES = 32
TILE_E = E // NTILES

MC = 40
MCP = 48
MNCH = TILE_E // MC
MG = MCP // 16

SB = 2000
NSB = TILE_E // SB
SGRP = SB // 16

_SCP = pltpu.CompilerParams(needs_layout_passes=False)
_MESH = plsc.VectorSubcoreMesh(core_axis_name="c", subcore_axis_name="s")


def _mask_constants():
    perm = jax.random.permutation(jax.random.key(42), N)
    mask_idx = perm[:NM]
    inv = jnp.argsort(perm)
    visf = (inv >= NM).astype(jnp.float32)
    pos_m = jax.random.normal(jax.random.key(7), (NM, 3), jnp.float32)
    pm_full = jnp.concatenate(
        [jnp.pad(pos_m, ((0, 0), (0, PW - 3))), jnp.zeros((NM, PW))], axis=0)
    posm8 = pm_full[inv]
    vis8 = jnp.pad(visf[:, None], ((0, 0), (0, PW - 1)))
    return mask_idx, visf, posm8, vis8


def _tc_node_pre(x_ref, w1_ref, w2_ref, a_ref, b_ref):
    xv = x_ref[...]
    a_ref[...] = jnp.dot(xv, w1_ref[...], preferred_element_type=jnp.float32)
    b_ref[...] = jnp.dot(xv, w2_ref[...], preferred_element_type=jnp.float32)


def _node_pre(x, w1, w2):
    return pl.pallas_call(
        _tc_node_pre,
        grid=(NB,),
        in_specs=[pl.BlockSpec((BN, D), lambda i: (i, 0)),
                  pl.BlockSpec((D, D), lambda i: (0, 0)),
                  pl.BlockSpec((D, D), lambda i: (0, 0))],
        out_specs=[pl.BlockSpec((BN, D), lambda i: (i, 0)),
                   pl.BlockSpec((BN, D), lambda i: (i, 0))],
        out_shape=[jax.ShapeDtypeStruct((N, D), jnp.float32),
                   jax.ShapeDtypeStruct((N, D), jnp.float32)],
    )(x, w1, w2)


def _tc_edge_pre(ea_ref, w3e_ref, w3d_ref, be_ref, bd_ref, oe_ref, od_ref):
    eav = ea_ref[...]
    oe_ref[...] = jnp.dot(eav, w3e_ref[...], preferred_element_type=jnp.float32) + be_ref[...]
    od_ref[...] = jnp.dot(eav, w3d_ref[...], preferred_element_type=jnp.float32) + bd_ref[...]


def _edge_pre(edge_attr, w3e, w3d, be, bd):
    return pl.pallas_call(
        _tc_edge_pre,
        grid=(EBG,),
        in_specs=[pl.BlockSpec((BEB, DE), lambda i: (i, 0)),
                  pl.BlockSpec((DE, D), lambda i: (0, 0)),
                  pl.BlockSpec((DE, D), lambda i: (0, 0)),
                  pl.BlockSpec((1, D), lambda i: (0, 0)),
                  pl.BlockSpec((1, D), lambda i: (0, 0))],
        out_specs=[pl.BlockSpec((BEB, D), lambda i: (i, 0)),
                   pl.BlockSpec((BEB, D), lambda i: (i, 0))],
        out_shape=[jax.ShapeDtypeStruct((E, D), jnp.float32),
                   jax.ShapeDtypeStruct((E, D), jnp.float32)],
    )(edge_attr, w3e, w3d, be, bd)


def _tc_reduce(p_ref, o_ref):
    o_ref[...] = jnp.sum(p_ref[...], axis=0, keepdims=True)


def _reduce_pacc(pacc_p):
    seg = PW * N // NB
    return pl.pallas_call(
        _tc_reduce,
        grid=(NB,),
        in_specs=[pl.BlockSpec((NTILES, seg), lambda i: (0, i))],
        out_specs=[pl.BlockSpec((1, seg), lambda i: (0, i))],
        out_shape=[jax.ShapeDtypeStruct((1, PW * N), jnp.float32)],
    )(pacc_p)[0].reshape(N, PW)


def _tc_enc_upd(x_ref, a0, a1, pacc_ref, pos_ref, vis_ref, posm_ref, tok_ref,
                wu1_ref, wu2_ref, bu_ref, w1d_ref, w2d_ref,
                a2_ref, b2_ref, pc_ref):
    agg = a0[...] + a1[...]
    pacc = pacc_ref[...]
    deg = pacc[:, 3:4] + 1.0
    h = jnp.maximum(
        jnp.dot(x_ref[...], wu1_ref[...], preferred_element_type=jnp.float32)
        + jnp.dot(agg / deg, wu2_ref[...], preferred_element_type=jnp.float32)
        + bu_ref[...], 0.0)
    vis = vis_ref[:, 0:1] > 0.0
    z = jnp.where(vis, h, tok_ref[...])
    pc_ref[...] = jnp.where(vis, pos_ref[...] + pacc / deg, posm_ref[...])
    a2_ref[...] = jnp.dot(z, w1d_ref[...], preferred_element_type=jnp.float32)
    b2_ref[...] = jnp.dot(z, w2d_ref[...], preferred_element_type=jnp.float32)


def _enc_upd(x, a0, a1, pacc, pos8, vis8, posm8, tok, wu1, wu2, bu, w1d, w2d):
    nd = lambda i: (i, 0)
    w0 = lambda i: (0, 0)
    return pl.pallas_call(
        _tc_enc_upd,
        grid=(NB,),
        in_specs=[pl.BlockSpec((BN, D), nd), pl.BlockSpec((BN, D), nd),
                  pl.BlockSpec((BN, D), nd), pl.BlockSpec((BN, PW), nd),
                  pl.BlockSpec((BN, PW), nd), pl.BlockSpec((BN, PW), nd),
                  pl.BlockSpec((BN, PW), nd),
                  pl.BlockSpec((1, D), w0),
                  pl.BlockSpec((D, D), w0), pl.BlockSpec((D, D), w0),
                  pl.BlockSpec((1, D), w0),
                  pl.BlockSpec((D, D), w0), pl.BlockSpec((D, D), w0)],
        out_specs=[pl.BlockSpec((BN, D), nd), pl.BlockSpec((BN, D), nd),
                   pl.BlockSpec((BN, PW), nd)],
        out_shape=[jax.ShapeDtypeStruct((N, D), jnp.float32),
                   jax.ShapeDtypeStruct((N, D), jnp.float32),
                   jax.ShapeDtypeStruct((N, PW), jnp.float32)],
    )(x, a0, a1, pacc, pos8, vis8, posm8, tok, wu1, wu2, bu, w1d, w2d)


def _tc_final(a0, a1, pacc_ref, tok_ref, wu1_ref, wu2_ref, bu_ref, wo_ref,
              bo_ref, posm_ref, out_ref):
    agg = a0[...] + a1[...]
    pacc = pacc_ref[...]
    deg = pacc[:, 3:4] + 1.0
    hz = jnp.maximum(
        jnp.dot(tok_ref[...], wu1_ref[...], preferred_element_type=jnp.float32)
        + jnp.dot(agg / deg, wu2_ref[...], preferred_element_type=jnp.float32)
        + bu_ref[...], 0.0)
    out_ref[...] = (jnp.dot(hz, wo_ref[...], preferred_element_type=jnp.float32)
                    + bo_ref[...] + posm_ref[...] + pacc / deg)


def _final(a0, a1, pacc, tok, wu1, wu2, bu, wo8, bo8, posm8):
    nd = lambda i: (i, 0)
    w0 = lambda i: (0, 0)
    return pl.pallas_call(
        _tc_final,
        grid=(NB,),
        in_specs=[pl.BlockSpec((BN, D), nd), pl.BlockSpec((BN, D), nd),
                  pl.BlockSpec((BN, PW), nd),
                  pl.BlockSpec((1, D), w0),
                  pl.BlockSpec((D, D), w0), pl.BlockSpec((D, D), w0),
                  pl.BlockSpec((1, D), w0),
                  pl.BlockSpec((D, PW), w0), pl.BlockSpec((1, PW), w0),
                  pl.BlockSpec((BN, PW), nd)],
        out_specs=[pl.BlockSpec((BN, PW), nd)],
        out_shape=[jax.ShapeDtypeStruct((N, PW), jnp.float32)],
    )(a0, a1, pacc, tok, wu1, wu2, bu, wo8, bo8, posm8)[0]


@functools.partial(
    pl.kernel,
    out_type=jax.ShapeDtypeStruct((E,), jnp.float32),
    mesh=_MESH,
    scratch_types=[
        pltpu.VMEM((N,), jnp.float32),
        pltpu.VMEM((N,), jnp.float32),
        pltpu.VMEM((N,), jnp.float32),
        pltpu.VMEM((N,), jnp.float32),
        pltpu.VMEM((N,), jnp.float32),
        pltpu.VMEM((SB,), jnp.int32),
        pltpu.VMEM((SB,), jnp.int32),
        pltpu.VMEM((SB,), jnp.float32),
    ],
    compiler_params=_SCP)
def _sc_pre(fs_h, fd_h, px_h, py_h, pz_h, src_h, dst_h, d2k_h,
            fs_v, fd_v, px_v, py_v, pz_v, sbig, dbig, obig):
    cid = lax.axis_index("c")
    sid = lax.axis_index("s")
    wid = cid * 16 + sid
    pltpu.sync_copy(fs_h, fs_v)
    pltpu.sync_copy(fd_h, fd_v)
    pltpu.sync_copy(px_h, px_v)
    pltpu.sync_copy(py_h, py_v)
    pltpu.sync_copy(pz_h, pz_v)

    def stage_body(t, carry):
        base = wid * TILE_E + t * SB
        pltpu.sync_copy(src_h.at[pl.ds(base, SB)], sbig)
        pltpu.sync_copy(dst_h.at[pl.ds(base, SB)], dbig)

        def grp(g, carry2):
            sv = sbig[pl.ds(g * 16, 16)]
            dv = dbig[pl.ds(g * 16, 16)]
            kf = plsc.load_gather(fs_v, [sv]) * plsc.load_gather(fd_v, [dv])
            dx = plsc.load_gather(px_v, [sv]) - plsc.load_gather(px_v, [dv])
            dy = plsc.load_gather(py_v, [sv]) - plsc.load_gather(py_v, [dv])
            dz = plsc.load_gather(pz_v, [sv]) - plsc.load_gather(pz_v, [dv])
            d2 = dx * dx + dy * dy + dz * dz
            obig[pl.ds(g * 16, 16)] = jnp.where(kf > 0.0, d2, -1.0)
            return carry2

        lax.fori_loop(0, SGRP, grp, 0)
        pltpu.sync_copy(obig, d2k_h.at[pl.ds(base, SB)])
        return carry

    lax.fori_loop(0, NSB, stage_body, 0)


_MAIN_OUT = [jax.ShapeDtypeStruct((N, D), jnp.float32),
             jax.ShapeDtypeStruct((N, D), jnp.float32),
             jax.ShapeDtypeStruct((E,), jnp.float32)]

_ZROWS = 624


@functools.partial(
    pl.kernel,
    out_type=_MAIN_OUT,
    mesh=_MESH,
    scratch_types=[
        pltpu.VMEM_SHARED((N, D), jnp.float32),
        [pltpu.VMEM((1, MCP), jnp.int32)] * 2,
        [pltpu.VMEM((1, MCP), jnp.int32)] * 2,
        [pltpu.VMEM((1, MCP + 16), jnp.float32)] * 2,
        pltpu.VMEM((1, MCP + 16), jnp.float32),
        pltpu.VMEM((1, MCP), jnp.int32),
        pltpu.VMEM((10 * MC + 16,), jnp.float32),
        pltpu.VMEM((MCP,), jnp.float32),
        [pltpu.VMEM((MCP, D), jnp.float32)] * 2,
        [pltpu.VMEM((MCP, D), jnp.float32)] * 2,
        [pltpu.VMEM((MCP, D), jnp.float32)] * 2,
        pltpu.VMEM((D,), jnp.float32),
        pltpu.VMEM((D,), jnp.float32),
        pltpu.VMEM((16, D), jnp.float32),
        [pltpu.SemaphoreType.DMA] * 2,
        [pltpu.SemaphoreType.DMA] * 2,
    ],
    compiler_params=_SCP)
def _sc_main(a_h, b_h, ea_h, d2k_h, src_h, dst_h, wr_h, wc_h,
             agg0_h, agg1_h, ct_h,
             agg_sh, sidx, didx, kbuf, kcur, dmrow, cbig, dbuf,
             arows, brows, erows, wr_v, wc_v, zb, gsem, isem):
    cid = lax.axis_index("c")
    sid = lax.axis_index("s")
    wid = cid * 16 + sid
    tbase = wid * TILE_E
    pltpu.sync_copy(wr_h, wr_v)
    pltpu.sync_copy(wc_h, wc_v)

    zvec = jnp.zeros((16,), jnp.float32)
    izero = jnp.zeros((16,), jnp.int32)
    for r in range(16):
        for q in range(D // 16):
            zb[r, pl.ds(q * 16, 16)] = zvec
    for s in range(2):
        didx[s][0, pl.ds(32, 16)] = izero
        sidx[s][0, pl.ds(32, 16)] = izero
        kbuf[s][0, pl.ds(32, 16)] = zvec - 1.0
        kbuf[s][0, pl.ds(48, 16)] = zvec - 1.0
        for r in range(MC, MCP):
            for q in range(D // 16):
                erows[s][r, pl.ds(q * 16, 16)] = zvec

    off = sid * _ZROWS
    nz = jnp.where(sid == 15, 40, 39)

    def zbody(i, carry):
        pltpu.sync_copy(zb, agg_sh.at[pl.ds(off + i * 16, 16)])
        return carry

    lax.fori_loop(0, nz, zbody, 0)
    plsc.subcore_barrier()

    lanes = jnp.arange(16, dtype=jnp.int32)

    def load_idx(c, s):
        base = tbase + c * MC
        pltpu.async_copy(src_h.at[pl.ds(base, MC)], sidx[s].at[0, pl.ds(0, MC)], isem[s])
        pltpu.async_copy(dst_h.at[pl.ds(base, MC)], didx[s].at[0, pl.ds(0, MC)], isem[s])
        pltpu.async_copy(d2k_h.at[pl.ds(base, MC)], kbuf[s].at[0, pl.ds(0, MC)], isem[s])

    def wait_idx(s):
        pltpu.make_async_copy(src_h.at[pl.ds(0, MC)], sidx[s].at[0, pl.ds(0, MC)], isem[s]).wait()
        pltpu.make_async_copy(dst_h.at[pl.ds(0, MC)], didx[s].at[0, pl.ds(0, MC)], isem[s]).wait()
        pltpu.make_async_copy(d2k_h.at[pl.ds(0, MC)], kbuf[s].at[0, pl.ds(0, MC)], isem[s]).wait()

    def issue_gathers(c, s):
        base = tbase + c * MC
        pltpu.async_copy(a_h.at[sidx[s].at[0]], arows[s], gsem[s])
        pltpu.async_copy(b_h.at[didx[s].at[0]], brows[s], gsem[s])
        pltpu.async_copy(ea_h.at[pl.ds(base, MC)], erows[s].at[pl.ds(0, MC)], gsem[s])

    def wait_gathers(s):
        pltpu.make_async_copy(a_h.at[sidx[s].at[0]], arows[s], gsem[s]).wait()
        pltpu.make_async_copy(b_h.at[didx[s].at[0]], brows[s], gsem[s]).wait()
        pltpu.make_async_copy(ea_h.at[pl.ds(0, MC)], erows[s].at[pl.ds(0, MC)], gsem[s]).wait()

    wks = [wr_v[pl.ds(16 * k, 16)] for k in range(D // 16)]
    wcs = [wc_v[pl.ds(16 * k, 16)] for k in range(D // 16)]

    lane0 = lanes == 0

    def compute_chunk(c, s):
        for g in range(MG):
            dmrow[0, pl.ds(g * 16, 16)] = didx[s][0, pl.ds(g * 16, 16)]
        for g in range(MG + 1):
            kcur[0, pl.ds(g * 16, 16)] = kbuf[s][0, pl.ds(g * 16, 16)]
        load_idx(jnp.minimum(c + 2, MNCH - 1), s)

        @plsc.parallel_loop(0, MCP, unroll=4)
        def ebody(e):
            kv = kcur[0, pl.ds(e, 16)][0]
            keep = jnp.where(kv >= 0.0, 1.0, 0.0)
            d2 = jnp.maximum(kv, 0.0)
            dotv = jnp.zeros((16,), jnp.float32)
            for k in range(D // 16):
                a = arows[s][e, pl.ds(16 * k, 16)]
                b = brows[s][e, pl.ds(16 * k, 16)]
                ee = erows[s][e, pl.ds(16 * k, 16)]
                m = jnp.maximum(a + b + ee + d2 * wks[k], 0.0) * keep
                arows[s][e, pl.ds(16 * k, 16)] = m
                dotv = dotv + m * wcs[k]
            dot = jnp.sum(dotv)
            plsc.store_scatter(dbuf, [izero + e], jnp.zeros((16,), jnp.float32) + dot,
                               mask=lane0)
        roff = lax.rem(c, 10) * MC
        for g in range(MG):
            kv = kcur[0, pl.ds(g * 16, 16)]
            keepv = jnp.where(kv >= 0.0, 1.0, 0.0)
            tv = jnp.exp(dbuf[pl.ds(g * 16, 16)] * 2.0)
            cbig[pl.ds(roff + g * 16, 16)] = (1.0 - 2.0 / (tv + 1.0)) * keepv

        @pl.when(lax.rem(c, 10) == 9)
        def _():
            pltpu.sync_copy(cbig.at[pl.ds(0, 10 * MC)],
                            ct_h.at[pl.ds(tbase + (c - 9) * MC, 10 * MC)])

        pltpu.sync_copy(arows[s], agg_sh.at[dmrow.at[0]], add=True)

    load_idx(0, 0)
    wait_idx(0)
    issue_gathers(0, 0)
    load_idx(1, 1)
    wait_idx(1)
    last = MNCH - 1

    def pair_body(i, carry):
        c0 = 2 * i
        c1 = c0 + 1
        issue_gathers(c1, 1)
        wait_gathers(0)
        compute_chunk(c0, 0)
        wait_idx(0)
        issue_gathers(jnp.minimum(c0 + 2, last), 0)
        wait_gathers(1)
        compute_chunk(c1, 1)
        wait_idx(1)
        return carry

    lax.fori_loop(0, MNCH // 2, pair_body, 0)
    wait_gathers(0)
    plsc.subcore_barrier()

    @pl.when(sid < 15)
    def _():
        @pl.when(cid == 0)
        def _():
            pltpu.sync_copy(agg_sh.at[pl.ds(off, _ZROWS)], agg0_h.at[pl.ds(off, _ZROWS)])
        @pl.when(cid == 1)
        def _():
            pltpu.sync_copy(agg_sh.at[pl.ds(off, _ZROWS)], agg1_h.at[pl.ds(off, _ZROWS)])

    @pl.when(sid == 15)
    def _():
        @pl.when(cid == 0)
        def _():
            pltpu.sync_copy(agg_sh.at[pl.ds(off, 640)], agg0_h.at[pl.ds(off, 640)])
        @pl.when(cid == 1)
        def _():
            pltpu.sync_copy(agg_sh.at[pl.ds(off, 640)], agg1_h.at[pl.ds(off, 640)])


@functools.partial(
    pl.kernel,
    out_type=jax.ShapeDtypeStruct((NTILES, PW * N), jnp.float32),
    mesh=_MESH,
    scratch_types=[
        pltpu.VMEM((N,), jnp.float32),
        pltpu.VMEM((N,), jnp.float32),
        pltpu.VMEM((N,), jnp.float32),
        pltpu.VMEM((PW * N,), jnp.float32),
        pltpu.VMEM((SB,), jnp.int32),
        pltpu.VMEM((SB,), jnp.int32),
        pltpu.VMEM((SB,), jnp.float32),
        pltpu.VMEM((SB,), jnp.float32),
    ],
    compiler_params=_SCP)
def _sc_post(px_h, py_h, pz_h, src_h, dst_h, d2k_h, ct_h, pacc_h,
             px_v, py_v, pz_v, pacc_v, sbig, dbig, kbig, cbig):
    cid = lax.axis_index("c")
    sid = lax.axis_index("s")
    wid = cid * 16 + sid
    pltpu.sync_copy(px_h, px_v)
    pltpu.sync_copy(py_h, py_v)
    pltpu.sync_copy(pz_h, pz_v)

    zvec = jnp.zeros((16,), jnp.float32)

    def zb(i, carry):
        pacc_v[pl.ds(i * 16, 16)] = zvec
        return carry

    lax.fori_loop(0, PW * N // 16, zb, 0)

    def stage_body(t, carry):
        base = wid * TILE_E + t * SB
        pltpu.sync_copy(src_h.at[pl.ds(base, SB)], sbig)
        pltpu.sync_copy(dst_h.at[pl.ds(base, SB)], dbig)
        pltpu.sync_copy(d2k_h.at[pl.ds(base, SB)], kbig)
        pltpu.sync_copy(ct_h.at[pl.ds(base, SB)], cbig)

        def grp(g, carry2):
            sv = sbig[pl.ds(g * 16, 16)]
            dv = dbig[pl.ds(g * 16, 16)]
            kv = kbig[pl.ds(g * 16, 16)]
            ct = cbig[pl.ds(g * 16, 16)]
            kf = jnp.where(kv >= 0.0, 1.0, 0.0)
            dx = plsc.load_gather(px_v, [sv]) - plsc.load_gather(px_v, [dv])
            dy = plsc.load_gather(py_v, [sv]) - plsc.load_gather(py_v, [dv])
            dz = plsc.load_gather(pz_v, [sv]) - plsc.load_gather(pz_v, [dv])
            dj = dv * PW
            plsc.addupdate_scatter(pacc_v, [dj], dx * ct)
            plsc.addupdate_scatter(pacc_v, [dj + 1], dy * ct)
            plsc.addupdate_scatter(pacc_v, [dj + 2], dz * ct)
            plsc.addupdate_scatter(pacc_v, [dj + 3], kf)
            return carry2

        lax.fori_loop(0, SGRP, grp, 0)
        return carry

    lax.fori_loop(0, NSB, stage_body, 0)
    pltpu.sync_copy(pacc_v, pacc_h.at[wid])


def _edge_phase(visf_s, visf_d, a_t, b_t, ea_t, px, py, pz, src, dst,
                wr_flat, wc_flat):
    d2k = _sc_pre(visf_s, visf_d, px, py, pz, src, dst)
    agg0, agg1, ct = _sc_main(a_t, b_t, ea_t, d2k, src, dst, wr_flat, wc_flat)
    pacc_p = _sc_post(px, py, pz, src, dst, d2k, ct)
    pacc = _reduce_pacc(pacc_p.reshape(NTILES, PW * N))
    return agg0, agg1, pacc


def kernel(x, pos, edge_index, edge_attr, batch_indices, masked_token,
           enc_W_msg, enc_b_msg, enc_W_upd, enc_b_upd, enc_w_coord,
           dec_W_msg, dec_b_msg, dec_W_upd, dec_b_upd, dec_w_coord,
           dec_W_out, dec_b_out):
    del batch_indices
    src = edge_index[0]
    dst = edge_index[1]

    w1e, w2e, w3e = enc_W_msg[:D], enc_W_msg[D:2 * D], enc_W_msg[2 * D:2 * D + DE]
    w1d, w2d, w3d = dec_W_msg[:D], dec_W_msg[D:2 * D], dec_W_msg[2 * D:2 * D + DE]
    wr_e = enc_W_msg[2 * D + DE]
    wr_d = dec_W_msg[2 * D + DE]
    wc_e = enc_w_coord[:, 0]
    wc_d = dec_w_coord[:, 0]
    wu1e, wu2e = enc_W_upd[:D], enc_W_upd[D:]
    wu1d, wu2d = dec_W_upd[:D], dec_W_upd[D:]
    be = enc_b_msg[None, :]
    bd = dec_b_msg[None, :]
    bue = enc_b_upd[None, :]
    bud = dec_b_upd[None, :]

    pos8 = jnp.pad(pos, ((0, 0), (0, PW - 3)))
    mask_idx, visf, posm8, vis8 = _mask_constants()
    maskf = 1.0 - visf
    onesf = jnp.ones((N,), jnp.float32)

    a_t, b_t = _node_pre(x, w1e, w2e)
    ea_e, ea_d = _edge_pre(edge_attr, w3e, w3d, be, bd)

    agg0, agg1, pacc = _edge_phase(
        visf, visf, a_t, b_t, ea_e, pos[:, 0], pos[:, 1], pos[:, 2],
        src, dst, wr_e, wc_e)

    a2, b2, posc8 = _enc_upd(x, agg0, agg1, pacc, pos8, vis8,
                             posm8, masked_token, wu1e, wu2e, bue, w1d, w2d)

    agg20, agg21, pacc2 = _edge_phase(
        onesf, maskf, a2, b2, ea_d, posc8[:, 0], posc8[:, 1], posc8[:, 2],
        src, dst, wr_d, wc_d)

    wo8 = jnp.pad(dec_W_out, ((0, 0), (0, PW - 3)))
    bo8 = jnp.pad(dec_b_out, (0, PW - 3))[None, :]
    rec8 = _final(agg20, agg21, pacc2, masked_token,
                  wu1d, wu2d, bud, wo8, bo8, posm8)

    return rec8[mask_idx, :3], mask_idx
